# baseline (speedup 1.0000x reference)
import functools

import jax
import jax.numpy as jnp
from jax import lax
from jax.experimental import pallas as pl
from jax.experimental.pallas import tpu as pltpu
from jax.experimental.pallas import tpu_sc as plsc

B = 1024
D_IN = 128
K = 16384
M = 32768
M_OUT = 64

_NC = 2
_NS = 16
_NW = _NC * _NS
_CHUNK = 128
_CHUNKS_PER_W = K // (_NW * _CHUNK)


def _sc_gather_body(u_hbm, idx_hbm, o_hbm, idx_v, rows_v, sem):
    wid = lax.axis_index("s") * _NC + lax.axis_index("c")
    base = wid * _CHUNKS_PER_W
    pltpu.sync_copy(idx_hbm.at[pl.ds(base, _CHUNKS_PER_W)], idx_v)
    copies = []
    for j in range(_CHUNKS_PER_W):
        copies.append(pltpu.async_copy(u_hbm.at[idx_v.at[j]], rows_v.at[j], sem))
    for c in copies:
        c.wait()
    pltpu.sync_copy(rows_v, o_hbm.at[pl.ds(base, _CHUNKS_PER_W)])


def _sc_gather(u_cat, idx2d):
    mesh = plsc.VectorSubcoreMesh(core_axis_name="c", subcore_axis_name="s")
    n_rows = K // _CHUNK
    run = pl.kernel(
        _sc_gather_body,
        mesh=mesh,
        out_type=jax.ShapeDtypeStruct((n_rows, _CHUNK, 2 * M_OUT), jnp.float32),
        scratch_types=[
            pltpu.VMEM((_CHUNKS_PER_W, _CHUNK), jnp.int32),
            pltpu.VMEM((_CHUNKS_PER_W, _CHUNK, 2 * M_OUT), jnp.float32),
            pltpu.SemaphoreType.DMA,
        ],
    )
    return run(u_cat, idx2d)


_CROWS = 4096


def _concat_body(re_ref, im_ref, out_ref):
    out_ref[:, :M_OUT] = re_ref[...]
    out_ref[:, M_OUT:] = im_ref[...]


def _tc_concat(u_re, u_im):
    return pl.pallas_call(
        _concat_body,
        grid=(M // _CROWS,),
        in_specs=[
            pl.BlockSpec((_CROWS, M_OUT), lambda i: (i, 0)),
            pl.BlockSpec((_CROWS, M_OUT), lambda i: (i, 0)),
        ],
        out_specs=pl.BlockSpec((_CROWS, 2 * M_OUT), lambda i: (i, 0)),
        out_shape=jax.ShapeDtypeStruct((M, 2 * M_OUT), jnp.float32),
    )(u_re, u_im)


_TK = 2048

_NT = (((1,), (1,)), ((), ()))
_NN = (((1,), (0,)), ((), ()))
_dotf = functools.partial(lax.dot_general,
                          precision=lax.Precision.DEFAULT,
                          preferred_element_type=jnp.float32)


def _gram_body(x_ref, wre_ref, wim_ref, n2_ref, g_acc):
    i = pl.program_id(0)

    @pl.when(i == 0)
    def _init():
        g_acc[...] = jnp.zeros_like(g_acc)

    g_acc[...] += _dotf(wre_ref[...], wre_ref[...], _NT)
    g_acc[...] += _dotf(wim_ref[...], wim_ref[...], _NT)

    @pl.when(i == pl.num_programs(0) - 1)
    def _final():
        xv = x_ref[...]
        t = _dotf(xv, g_acc[...], _NN)
        n2_ref[...] = jnp.maximum(jnp.sum(t * xv, axis=1), 0.0)


def _tc_gram(x, w_re, w_im):
    grid = K // _TK
    return pl.pallas_call(
        _gram_body,
        grid=(grid,),
        in_specs=[
            pl.BlockSpec((B, D_IN), lambda i: (0, 0)),
            pl.BlockSpec((D_IN, _TK), lambda i: (0, i)),
            pl.BlockSpec((D_IN, _TK), lambda i: (0, i)),
        ],
        out_specs=pl.BlockSpec((B,), lambda i: (0,)),
        out_shape=jax.ShapeDtypeStruct((B,), jnp.float32),
        scratch_shapes=[
            pltpu.VMEM((D_IN, D_IN), jnp.float32),
        ],
    )(x, w_re, w_im)


def _readout_body(x_ref, wre_ref, wim_ref, ugcat_ref, n2_ref, out_ref,
                  dre_acc, dim_acc):
    i = pl.program_id(0)

    @pl.when(i == 0)
    def _init():
        dre_acc[...] = jnp.zeros_like(dre_acc)
        dim_acc[...] = jnp.zeros_like(dim_acc)

    dre_acc[...] += _dotf(wre_ref[...], ugcat_ref[...], _NN)
    dim_acc[...] += _dotf(wim_ref[...], ugcat_ref[...], _NN)

    @pl.when(i == pl.num_programs(0) - 1)
    def _final():
        xv = x_ref[...]
        d_re = dre_acc[...]
        d_im = dim_acc[...]
        c_re = d_re[:, :M_OUT] - d_im[:, M_OUT:]
        c_im = d_re[:, M_OUT:] + d_im[:, :M_OUT]
        norm = jnp.sqrt(n2_ref[...]) + 1e-20
        o_re = _dotf(xv, c_re, _NN)
        o_im = _dotf(xv, c_im, _NN)
        out_ref[...] = (o_re * o_re + o_im * o_im) / (norm * norm)[:, None]


def _tc_readout(x, w_re, w_im, ug_cat, n2):
    grid = K // _TK
    return pl.pallas_call(
        _readout_body,
        grid=(grid,),
        in_specs=[
            pl.BlockSpec((B, D_IN), lambda i: (0, 0)),
            pl.BlockSpec((D_IN, _TK), lambda i: (0, i)),
            pl.BlockSpec((D_IN, _TK), lambda i: (0, i)),
            pl.BlockSpec((_TK, 2 * M_OUT), lambda i: (i, 0)),
            pl.BlockSpec((B,), lambda i: (0,)),
        ],
        out_specs=pl.BlockSpec((B, M_OUT), lambda i: (0, 0)),
        out_shape=jax.ShapeDtypeStruct((B, M_OUT), jnp.float32),
        scratch_shapes=[
            pltpu.VMEM((D_IN, 2 * M_OUT), jnp.float32),
            pltpu.VMEM((D_IN, 2 * M_OUT), jnp.float32),
        ],
    )(x, w_re, w_im, ug_cat, n2)


def kernel(x, W_re, W_im, U_re, U_im, index_map):
    idx2d = index_map.astype(jnp.int32).reshape(K // _CHUNK, _CHUNK)
    u_cat = jnp.concatenate([U_re, U_im], axis=1)
    ug = _sc_gather(u_cat, idx2d)
    ug_cat = ug.reshape(K, 2 * M_OUT)
    n2 = _tc_gram(x, W_re, W_im)
    return _tc_readout(x, W_re, W_im, ug_cat, n2)


# speedup vs baseline: 1.0826x; 1.0826x over previous
import functools

import jax
import jax.numpy as jnp
from jax import lax
from jax.experimental import pallas as pl
from jax.experimental.pallas import tpu as pltpu
from jax.experimental.pallas import tpu_sc as plsc

B = 1024
D_IN = 128
K = 16384
M = 32768
M_OUT = 64

_NC = 2
_NS = 16
_NW = _NC * _NS
_CHUNK = 128
_CHUNKS_PER_W = K // (_NW * _CHUNK)


def _sc_gather_body(u_hbm, idx_hbm, o_hbm, idx_v, rows_v, sem, wsem):
    wid = lax.axis_index("s") * _NC + lax.axis_index("c")
    base = wid * _CHUNKS_PER_W
    pltpu.sync_copy(idx_hbm.at[pl.ds(base, _CHUNKS_PER_W)], idx_v)
    gathers = [pltpu.async_copy(u_hbm.at[idx_v.at[j]], rows_v.at[j], sem)
               for j in range(_CHUNKS_PER_W)]
    writes = []
    for j in range(_CHUNKS_PER_W):
        gathers[j].wait()
        writes.append(pltpu.async_copy(rows_v.at[j], o_hbm.at[base + j], wsem))
    for w in writes:
        w.wait()


def _sc_gather(u_cat, idx2d):
    mesh = plsc.VectorSubcoreMesh(core_axis_name="c", subcore_axis_name="s")
    n_rows = K // _CHUNK
    run = pl.kernel(
        _sc_gather_body,
        mesh=mesh,
        out_type=jax.ShapeDtypeStruct((n_rows, _CHUNK, 2 * M_OUT), jnp.float32),
        scratch_types=[
            pltpu.VMEM((_CHUNKS_PER_W, _CHUNK), jnp.int32),
            pltpu.VMEM((_CHUNKS_PER_W, _CHUNK, 2 * M_OUT), jnp.float32),
            pltpu.SemaphoreType.DMA,
            pltpu.SemaphoreType.DMA,
        ],
    )
    return run(u_cat, idx2d)


_CROWS = 4096


def _concat_body(re_ref, im_ref, out_ref):
    out_ref[:, :M_OUT] = re_ref[...]
    out_ref[:, M_OUT:] = im_ref[...]


def _tc_concat(u_re, u_im):
    return pl.pallas_call(
        _concat_body,
        grid=(M // _CROWS,),
        in_specs=[
            pl.BlockSpec((_CROWS, M_OUT), lambda i: (i, 0)),
            pl.BlockSpec((_CROWS, M_OUT), lambda i: (i, 0)),
        ],
        out_specs=pl.BlockSpec((_CROWS, 2 * M_OUT), lambda i: (i, 0)),
        out_shape=jax.ShapeDtypeStruct((M, 2 * M_OUT), jnp.float32),
    )(u_re, u_im)


_TK = 4096

_NT = (((1,), (1,)), ((), ()))
_NN = (((1,), (0,)), ((), ()))
_dotf = functools.partial(lax.dot_general,
                          precision=lax.Precision.DEFAULT,
                          preferred_element_type=jnp.float32)


def _tc_body(x_ref, wre_ref, wim_ref, ugcat_ref, out_ref,
             g_acc, dre_acc, dim_acc):
    i = pl.program_id(0)

    @pl.when(i == 0)
    def _init():
        g_acc[...] = jnp.zeros_like(g_acc)
        dre_acc[...] = jnp.zeros_like(dre_acc)
        dim_acc[...] = jnp.zeros_like(dim_acc)

    wre = wre_ref[...]
    wim = wim_ref[...]
    ugcat = ugcat_ref[...]
    g_acc[...] += _dotf(wre, wre, _NT)
    g_acc[...] += _dotf(wim, wim, _NT)
    dre_acc[...] += _dotf(wre, ugcat, _NN)
    dim_acc[...] += _dotf(wim, ugcat, _NN)

    @pl.when(i == pl.num_programs(0) - 1)
    def _final():
        xv = x_ref[...]
        d_re = dre_acc[...]
        d_im = dim_acc[...]
        c_re = d_re[:, :M_OUT] - d_im[:, M_OUT:]
        c_im = d_re[:, M_OUT:] + d_im[:, :M_OUT]
        t = _dotf(xv, g_acc[...], _NN)
        n2 = jnp.maximum(jnp.sum(t * xv, axis=1, keepdims=True), 0.0)
        norm = jnp.sqrt(n2) + 1e-20
        o_re = _dotf(xv, c_re, _NN)
        o_im = _dotf(xv, c_im, _NN)
        out_ref[...] = (o_re * o_re + o_im * o_im) / (norm * norm)


def _tc_fused(x, w_re, w_im, ug_cat):
    grid = K // _TK
    return pl.pallas_call(
        _tc_body,
        grid=(grid,),
        in_specs=[
            pl.BlockSpec((B, D_IN), lambda i: (0, 0)),
            pl.BlockSpec((D_IN, _TK), lambda i: (0, i)),
            pl.BlockSpec((D_IN, _TK), lambda i: (0, i)),
            pl.BlockSpec((_TK, 2 * M_OUT), lambda i: (i, 0)),
        ],
        out_specs=pl.BlockSpec((B, M_OUT), lambda i: (0, 0)),
        out_shape=jax.ShapeDtypeStruct((B, M_OUT), jnp.float32),
        scratch_shapes=[
            pltpu.VMEM((D_IN, D_IN), jnp.float32),
            pltpu.VMEM((D_IN, 2 * M_OUT), jnp.float32),
            pltpu.VMEM((D_IN, 2 * M_OUT), jnp.float32),
        ],
    )(x, w_re, w_im, ug_cat)


def kernel(x, W_re, W_im, U_re, U_im, index_map):
    idx2d = index_map.astype(jnp.int32).reshape(K // _CHUNK, _CHUNK)
    u_cat = jnp.concatenate([U_re, U_im], axis=1)
    ug = _sc_gather(u_cat, idx2d)
    ug_cat = ug.reshape(K, 2 * M_OUT)
    return _tc_fused(x, W_re, W_im, ug_cat)
